# Initial kernel scaffold; baseline (speedup 1.0000x reference)
import jax
import jax.numpy as jnp
from jax.experimental import pallas as pl


def kernel(hidden_states, seq_start_end, curr_pos, W, b):
    raise NotImplementedError("write your pallas kernel here")


# speedup vs baseline: 5.9696x; 5.9696x over previous
import jax
import jax.numpy as jnp
from jax import lax
from jax.experimental import pallas as pl
from jax.experimental.pallas import tpu as pltpu

_H = 64
_EMB = 64
_G = 8
_NEIGH = 4.0
_NPED = 512
_NSEQ = 8


def _pool_kernel(seq_ref, hidden_ref, pos_ref, posT_ref, w3_ref, b_ref, out_ref):
    G = _G
    half = G // 2
    inv_gsn = (G - 1) / _NEIGH

    px_col = pos_ref[:, 0:1]
    py_col = pos_ref[:, 1:2]
    px_row = posT_ref[0:1, :]
    py_row = posT_ref[1:2, :]

    fx = jnp.clip((px_row - px_col) * inv_gsn, -half, half) + half
    fy = jnp.clip((py_row - py_col) * inv_gsn, -half, half) + half
    gx = fx.astype(jnp.int32)
    gy = fy.astype(jnp.int32)
    cell = gy * G + gx

    col_i = lax.broadcasted_iota(jnp.int32, (_NPED, 1), 0)
    row_j = lax.broadcasted_iota(jnp.int32, (1, _NPED), 1)

    m = jnp.zeros((_NPED, _NPED), dtype=jnp.float32)
    for s in range(_NSEQ):
        st = seq_ref[s, 0]
        en = seq_ref[s, 1]
        mi = ((col_i >= st) & (col_i < en)).astype(jnp.float32)
        mj = ((row_j >= st) & (row_j < en)).astype(jnp.float32)
        m = m + mi * mj

    valid = (gx < G) & (gy < G) & (col_i != row_j)
    w = jnp.where(valid, m, 0.0)

    hidden = hidden_ref[...]
    acc = jnp.zeros((_NPED, _EMB), dtype=jnp.float32)
    for c in range(G * G):
        a = jnp.where(cell == c, w, 0.0)
        p = jnp.dot(a, hidden, preferred_element_type=jnp.float32)
        wc = w3_ref[c]
        acc = acc + lax.dot_general(
            p, wc, (((1,), (0,)), ((), ())),
            preferred_element_type=jnp.float32)
    out_ref[...] = jnp.maximum(acc + b_ref[0:1, :], 0.0)


def kernel(hidden_states, seq_start_end, curr_pos, W, b):
    w3 = W.reshape(_EMB, _G * _G, _H).transpose(1, 2, 0)
    posT = curr_pos.T
    b2 = b.reshape(1, _EMB)
    seq = seq_start_end.astype(jnp.int32)

    return pl.pallas_call(
        _pool_kernel,
        out_shape=jax.ShapeDtypeStruct((_NPED, _EMB), jnp.float32),
        in_specs=[
            pl.BlockSpec(memory_space=pltpu.SMEM),
            pl.BlockSpec(memory_space=pltpu.VMEM),
            pl.BlockSpec(memory_space=pltpu.VMEM),
            pl.BlockSpec(memory_space=pltpu.VMEM),
            pl.BlockSpec(memory_space=pltpu.VMEM),
            pl.BlockSpec(memory_space=pltpu.VMEM),
        ],
        out_specs=pl.BlockSpec(memory_space=pltpu.VMEM),
    )(seq, hidden_states, curr_pos, posT, w3, b2)
